# baseline (speedup 1.0000x reference)
import functools

import jax
import jax.numpy as jnp
from jax import lax
from jax.experimental import pallas as pl
from jax.experimental.pallas import tpu as pltpu
from jax.experimental.pallas import tpu_sc as plsc

B = 16384
K = 8
D = 32
TOT = B * K
NW = 32
PER_W = TOT // NW
CHUNK = 128
NCH = PER_W // CHUNK

_mesh = plsc.VectorSubcoreMesh(core_axis_name="c", subcore_axis_name="s")


@functools.partial(
    pl.kernel,
    mesh=_mesh,
    out_type=jax.ShapeDtypeStruct((TOT, D), jnp.float32),
    scratch_types=[
        pltpu.VMEM((NCH, CHUNK), jnp.int32),
        pltpu.VMEM((2, CHUNK, D), jnp.float32),
        pltpu.SemaphoreType.DMA,
    ],
    compiler_params=pltpu.CompilerParams(use_tc_tiling_on_sc=False),
)
def _gather32(idx_hbm, table_hbm, out_hbm, idx_v, rows_v, sem):
    wid = lax.axis_index("s") * 2 + lax.axis_index("c")
    base = wid * PER_W
    pltpu.sync_copy(idx_hbm.at[pl.ds(wid * NCH, NCH)], idx_v)

    def body(j, _):
        buf = lax.rem(j, 2)
        pltpu.async_copy(
            table_hbm.at[idx_v.at[j]], rows_v.at[buf], sem
        ).wait()
        pltpu.sync_copy(
            rows_v.at[buf], out_hbm.at[pl.ds(base + j * CHUNK, CHUNK)]
        )
        return 0

    lax.fori_loop(0, NCH, body, 0)


def kernel(point_feats_idx, values_weight):
    idx = point_feats_idx.astype(jnp.int32).reshape(NW * NCH, CHUNK)
    flat = _gather32(idx, values_weight)
    return flat.reshape(B, K, D)


# speedup vs baseline: 1.0394x; 1.0394x over previous
import functools

import jax
import jax.numpy as jnp
from jax import lax
from jax.experimental import pallas as pl
from jax.experimental.pallas import tpu as pltpu
from jax.experimental.pallas import tpu_sc as plsc

B = 16384
K = 8
D = 32
TOT = B * K
NW = 32
PER_W = TOT // NW
CHUNK = 128
NCH = PER_W // CHUNK
NBUF = 8

_mesh = plsc.VectorSubcoreMesh(core_axis_name="c", subcore_axis_name="s")


@functools.partial(
    pl.kernel,
    mesh=_mesh,
    out_type=jax.ShapeDtypeStruct((TOT, D), jnp.float32),
    scratch_types=[
        pltpu.VMEM((NCH, CHUNK), jnp.int32),
        pltpu.VMEM((NBUF, CHUNK, D), jnp.float32),
    ]
    + [pltpu.SemaphoreType.DMA] * (2 * NBUF),
    compiler_params=pltpu.CompilerParams(use_tc_tiling_on_sc=False),
)
def _gather32(idx_hbm, table_hbm, out_hbm, idx_v, rows_v, *sems):
    gsems, osems = sems[:NBUF], sems[NBUF:]
    wid = lax.axis_index("s") * 2 + lax.axis_index("c")
    base = wid * PER_W
    pltpu.sync_copy(idx_hbm.at[pl.ds(wid * NCH, NCH)], idx_v)

    def gather(j, b):
        return pltpu.make_async_copy(
            table_hbm.at[idx_v.at[j]], rows_v.at[b], gsems[b])

    def writeback(j, b):
        return pltpu.make_async_copy(
            rows_v.at[b], out_hbm.at[pl.ds(base + j * CHUNK, CHUNK)],
            osems[b])

    for j in range(min(NBUF, NCH)):
        gather(j, j).start()
    for j in range(NCH):
        b = j % NBUF
        gather(j, b).wait()
        writeback(j, b).start()
        if j + NBUF < NCH:
            writeback(j, b).wait()
            gather(j + NBUF, b).start()
    for j in range(max(0, NCH - NBUF), NCH):
        writeback(j, j % NBUF).wait()


def kernel(point_feats_idx, values_weight):
    idx = point_feats_idx.astype(jnp.int32).reshape(NW * NCH, CHUNK)
    flat = _gather32(idx, values_weight)
    return flat.reshape(B, K, D)
